# Initial kernel scaffold; baseline (speedup 1.0000x reference)
import jax
import jax.numpy as jnp
from jax.experimental import pallas as pl


def kernel(pix_to_face, bary_coords, verts, faces, cam_origin):
    raise NotImplementedError("write your pallas kernel here")


# speedup vs baseline: 20.1523x; 20.1523x over previous
import functools

import jax
import jax.numpy as jnp
from jax import lax
from jax.experimental import pallas as pl
from jax.experimental.pallas import tpu as pltpu
from jax.experimental.pallas import tpu_sc as plsc

_NC = 2
_NS = 16
_NW = _NC * _NS
_L = 16

_FCB = 256
_PCB = 256


def _rsqrt(x):
    i = plsc.bitcast(x, jnp.int32)
    y = plsc.bitcast(jnp.int32(0x5F3759DF) - (i >> 1), jnp.float32)
    xh = x * 0.5
    for _ in range(3):
        y = y * (1.5 - xh * y * y)
    return y


def _col(c):
    return jnp.full((_L,), c, jnp.int32)


def _make_facetab_kernel(f_pad):
    nchunk = f_pad // (_NW * _FCB)
    mesh = plsc.VectorSubcoreMesh(core_axis_name="c", subcore_axis_name="s")

    @functools.partial(
        pl.kernel,
        out_type=jax.ShapeDtypeStruct((f_pad * 16,), jnp.float32),
        mesh=mesh,
        compiler_params=pltpu.CompilerParams(needs_layout_passes=False, use_tc_tiling_on_sc=False),
        scratch_types=[
            pltpu.VMEM((_FCB * 3,), jnp.int32),
            pltpu.VMEM((_FCB * 3, 16), jnp.float32),
            pltpu.VMEM((_FCB * 16,), jnp.float32),
            pltpu.SemaphoreType.DMA,
        ],
    )
    def facetab_kernel(verts_hbm, faces_hbm, ftab_hbm, fidx_v, vrows_v, fout_v, sem):
        wid = lax.axis_index("c") * _NS + lax.axis_index("s")
        iota = lax.iota(jnp.int32, _L)
        iota3 = iota * 3
        iota16 = iota * 16
        tec_base = wid * (nchunk * _FCB)

        def chunk_body(ch, carry):
            fbase = tec_base + ch * _FCB
            pltpu.sync_copy(faces_hbm.at[pl.ds(fbase * 3, _FCB * 3)], fidx_v)
            pltpu.async_copy(verts_hbm.at[fidx_v], vrows_v, sem).wait()

            def group_body(g, c2):
                v = []
                for j in range(3):
                    row = g * 48 + iota3 + j
                    v.append([plsc.load_gather(vrows_v, [row, _col(m)])
                              for m in range(3)])
                e1 = [v[1][m] - v[0][m] for m in range(3)]
                e2 = [v[2][m] - v[0][m] for m in range(3)]
                n = [e1[1] * e2[2] - e1[2] * e2[1],
                     e1[2] * e2[0] - e1[0] * e2[2],
                     e1[0] * e2[1] - e1[1] * e2[0]]
                len2 = jnp.maximum(n[0] * n[0] + n[1] * n[1] + n[2] * n[2],
                                   1e-24)
                r = _rsqrt(len2)
                obase = g * 256 + iota16
                for j in range(3):
                    for m in range(3):
                        plsc.store_scatter(fout_v, [obase + (j * 4 + m)],
                                           v[j][m])
                for m in range(3):
                    plsc.store_scatter(fout_v, [obase + (12 + m)], n[m] * r)
                return c2

            lax.fori_loop(0, _FCB // _L, group_body, 0)
            pltpu.sync_copy(fout_v, ftab_hbm.at[pl.ds(fbase * 16, _FCB * 16)])
            return carry

        lax.fori_loop(0, nchunk, chunk_body, 0)

    return facetab_kernel


def _make_shade_kernel(np_pix, f_pad, pix_per_batch):
    nchunk = np_pix // (_NW * _PCB)
    mesh = plsc.VectorSubcoreMesh(core_axis_name="c", subcore_axis_name="s")
    out = jax.ShapeDtypeStruct((np_pix,), jnp.float32)

    @functools.partial(
        pl.kernel,
        out_type=(out, out, out),
        mesh=mesh,
        compiler_params=pltpu.CompilerParams(needs_layout_passes=False, use_tc_tiling_on_sc=False),
        scratch_types=[
            pltpu.VMEM((_PCB * 3,), jnp.int32),
            pltpu.VMEM((_PCB * 9,), jnp.float32),
            pltpu.VMEM((_PCB * 3, 16), jnp.float32),
            pltpu.VMEM((_PCB,), jnp.float32),
            pltpu.VMEM((_PCB,), jnp.float32),
            pltpu.VMEM((_PCB,), jnp.float32),
            pltpu.VMEM((16,), jnp.float32),
            pltpu.SemaphoreType.DMA,
        ],
    )
    def shade_kernel(ftab_hbm, p2f_hbm, bary_hbm, cam_hbm, o0_hbm, o1_hbm,
                     o2_hbm, idx_v, bary_v, rows_v, o0_v, o1_v, o2_v, cam_v,
                     sem):
        wid = lax.axis_index("c") * _NS + lax.axis_index("s")
        iota = lax.iota(jnp.int32, _L)
        iota3 = iota * 3
        iota9 = iota * 9
        tec_base = wid * (nchunk * _PCB)
        pltpu.sync_copy(cam_hbm, cam_v)
        outs = (o0_v, o1_v, o2_v)
        out_hbms = (o0_hbm, o1_hbm, o2_hbm)

        def chunk_body(ch, carry):
            pbase = tec_base + ch * _PCB
            b = pbase // pix_per_batch
            zero16 = jnp.zeros((_L,), jnp.int32)
            cam = [plsc.load_gather(cam_v, [zero16 + (b * 3 + m)])
                   for m in range(3)]
            pltpu.sync_copy(p2f_hbm.at[pl.ds(pbase * 3, _PCB * 3)], idx_v)
            pltpu.sync_copy(bary_hbm.at[pl.ds(pbase * 9, _PCB * 9)], bary_v)
            pltpu.async_copy(ftab_hbm.at[idx_v], rows_v, sem).wait()

            def group_body(g, c2):
                for k in range(3):
                    row = g * 48 + iota3 + k
                    vv = [[plsc.load_gather(rows_v, [row, _col(j * 4 + m)])
                           for m in range(3)] for j in range(3)]
                    nn = [plsc.load_gather(rows_v, [row, _col(12 + m)])
                          for m in range(3)]
                    bbase = g * 144 + iota9 + k * 3
                    bb = [plsc.load_gather(bary_v, [bbase + j])
                          for j in range(3)]
                    pts = [bb[0] * vv[0][m] + bb[1] * vv[1][m]
                           + bb[2] * vv[2][m] for m in range(3)]
                    view = [pts[m] - cam[m] for m in range(3)]
                    len2 = jnp.maximum(view[0] * view[0] + view[1] * view[1]
                                       + view[2] * view[2], 1e-24)
                    r = _rsqrt(len2)
                    d = (nn[0] * view[0] + nn[1] * view[1]
                         + nn[2] * view[2]) * r
                    outs[k][pl.ds(g * 16, 16)] = d
                return c2

            lax.fori_loop(0, _PCB // _L, group_body, 0)
            for k in range(3):
                pltpu.sync_copy(outs[k], out_hbms[k].at[pl.ds(pbase, _PCB)])
            return carry

        lax.fori_loop(0, nchunk, chunk_body, 0)

    return shade_kernel


def kernel(pix_to_face, bary_coords, verts, faces, cam_origin):
    n, h, w, k = pix_to_face.shape
    np_pix = n * h * w
    v_cnt = verts.shape[0]
    f_cnt = faces.shape[0]
    align = _NW * _FCB
    f_pad = ((f_cnt + align - 1) // align) * align

    verts_pad = jnp.zeros((v_cnt, 16), jnp.float32).at[:, :3].set(verts)
    faces_flat = jnp.concatenate(
        [faces.reshape(-1),
         jnp.zeros((f_pad - f_cnt) * 3, jnp.int32)])
    p2f_flat = pix_to_face.reshape(-1)
    bary_flat = bary_coords.reshape(-1)
    cam_pad = jnp.zeros((16,), jnp.float32).at[: n * 3].set(
        cam_origin.reshape(-1))

    ftab = _make_facetab_kernel(f_pad)(verts_pad, faces_flat)
    o0, o1, o2 = _make_shade_kernel(np_pix, f_pad, h * w)(
        ftab.reshape(f_pad, 16), p2f_flat, bary_flat, cam_pad)
    return tuple(o.reshape(n, h, w, 1) for o in (o0, o1, o2))


# speedup vs baseline: 20.9610x; 1.0401x over previous
import functools

import jax
import jax.numpy as jnp
from jax import lax
from jax.experimental import pallas as pl
from jax.experimental.pallas import tpu as pltpu
from jax.experimental.pallas import tpu_sc as plsc

_NC = 2
_NS = 16
_NW = _NC * _NS
_L = 16

_FCB = 256
_PCB = 256

_PARAMS = pltpu.CompilerParams(needs_layout_passes=False,
                               use_tc_tiling_on_sc=False)


def _rsqrt(x):
    i = plsc.bitcast(x, jnp.int32)
    y = plsc.bitcast(jnp.int32(0x5F3759DF) - (i >> 1), jnp.float32)
    xh = x * 0.5
    for _ in range(3):
        y = y * (1.5 - xh * y * y)
    return y


def _col(c):
    return jnp.full((_L,), c, jnp.int32)


def _maybe(pred, fn):
    if isinstance(pred, bool):
        if pred:
            fn()
    else:
        pl.when(pred)(fn)


def _make_facetab_kernel(f_pad):
    nchunk = f_pad // (_NW * _FCB)
    mesh = plsc.VectorSubcoreMesh(core_axis_name="c", subcore_axis_name="s")

    @functools.partial(
        pl.kernel,
        out_type=jax.ShapeDtypeStruct((f_pad * 16,), jnp.float32),
        mesh=mesh,
        compiler_params=_PARAMS,
        scratch_types=[
            pltpu.VMEM((_FCB * 3,), jnp.int32),
            pltpu.VMEM((_FCB * 3,), jnp.int32),
            pltpu.VMEM((_FCB * 3, 16), jnp.float32),
            pltpu.VMEM((_FCB * 3, 16), jnp.float32),
            pltpu.VMEM((_FCB * 16,), jnp.float32),
            pltpu.SemaphoreType.DMA,
            pltpu.SemaphoreType.DMA,
            pltpu.SemaphoreType.DMA,
            pltpu.SemaphoreType.DMA,
        ],
    )
    def facetab_kernel(verts_hbm, faces_hbm, ftab_hbm, fidx0, fidx1, vrows0,
                       vrows1, fout_v, si0, si1, sg0, sg1):
        fidxs, vrows = (fidx0, fidx1), (vrows0, vrows1)
        sin, sg = (si0, si1), (sg0, sg1)
        wid = lax.axis_index("c") * _NS + lax.axis_index("s")
        iota = lax.iota(jnp.int32, _L)
        iota3 = iota * 3
        iota16 = iota * 16
        tec_base = wid * (nchunk * _FCB)

        def start_in(ch, b):
            fbase = tec_base + ch * _FCB
            pltpu.async_copy(faces_hbm.at[pl.ds(fbase * 3, _FCB * 3)],
                             fidxs[b], sin[b])

        def wait_in(b):
            pltpu.make_async_copy(faces_hbm.at[pl.ds(0, _FCB * 3)],
                                  fidxs[b], sin[b]).wait()

        def start_gather(b):
            pltpu.async_copy(verts_hbm.at[fidxs[b]], vrows[b], sg[b])

        def wait_gather(b):
            pltpu.make_async_copy(verts_hbm.at[fidxs[b]], vrows[b],
                                  sg[b]).wait()

        def do_chunk(ch, b, pred_next, pred_next2):
            q = 1 - b
            wait_gather(b)

            def _next():
                wait_in(q)
                start_gather(q)
            _maybe(pred_next, _next)

            def group_body(g, c2):
                v = []
                for j in range(3):
                    row = g * 48 + iota3 + j
                    v.append([plsc.load_gather(vrows[b], [row, _col(m)])
                              for m in range(3)])
                e1 = [v[1][m] - v[0][m] for m in range(3)]
                e2 = [v[2][m] - v[0][m] for m in range(3)]
                n = [e1[1] * e2[2] - e1[2] * e2[1],
                     e1[2] * e2[0] - e1[0] * e2[2],
                     e1[0] * e2[1] - e1[1] * e2[0]]
                len2 = jnp.maximum(n[0] * n[0] + n[1] * n[1] + n[2] * n[2],
                                   1e-24)
                r = _rsqrt(len2)
                obase = g * 256 + iota16
                for j in range(3):
                    for m in range(3):
                        plsc.store_scatter(fout_v, [obase + (j * 4 + m)],
                                           v[j][m])
                for m in range(3):
                    plsc.store_scatter(fout_v, [obase + (12 + m)], n[m] * r)
                return c2

            lax.fori_loop(0, _FCB // _L, group_body, 0)
            fbase = tec_base + ch * _FCB
            pltpu.sync_copy(fout_v, ftab_hbm.at[pl.ds(fbase * 16, _FCB * 16)])
            _maybe(pred_next2, lambda: start_in(ch + 2, b))

        start_in(0, 0)
        wait_in(0)
        start_gather(0)
        start_in(1, 1)

        def pair_body(cp, carry):
            for b in (0, 1):
                ch = cp * 2 + b
                do_chunk(ch, b, ch + 1 < nchunk, ch + 2 < nchunk)
            return carry

        lax.fori_loop(0, nchunk // 2, pair_body, 0)
        if nchunk % 2:
            do_chunk(nchunk - 1, (nchunk - 1) % 2, False, False)

    return facetab_kernel


def _make_shade_kernel(np_pix, f_pad, pix_per_batch):
    nchunk = np_pix // (_NW * _PCB)
    mesh = plsc.VectorSubcoreMesh(core_axis_name="c", subcore_axis_name="s")
    out = jax.ShapeDtypeStruct((np_pix,), jnp.float32)

    @functools.partial(
        pl.kernel,
        out_type=(out, out, out),
        mesh=mesh,
        compiler_params=_PARAMS,
        scratch_types=[
            pltpu.VMEM((_PCB * 3,), jnp.int32),
            pltpu.VMEM((_PCB * 3,), jnp.int32),
            pltpu.VMEM((_PCB * 9,), jnp.float32),
            pltpu.VMEM((_PCB * 9,), jnp.float32),
            pltpu.VMEM((_PCB * 3, 16), jnp.float32),
            pltpu.VMEM((_PCB * 3, 16), jnp.float32),
            pltpu.VMEM((_PCB,), jnp.float32),
            pltpu.VMEM((_PCB,), jnp.float32),
            pltpu.VMEM((_PCB,), jnp.float32),
            pltpu.VMEM((16,), jnp.float32),
            pltpu.SemaphoreType.DMA,
            pltpu.SemaphoreType.DMA,
            pltpu.SemaphoreType.DMA,
            pltpu.SemaphoreType.DMA,
        ],
    )
    def shade_kernel(ftab_hbm, p2f_hbm, bary_hbm, cam_hbm, o0_hbm, o1_hbm,
                     o2_hbm, idx0, idx1, bry0, bry1, rows0, rows1, o0_v, o1_v,
                     o2_v, cam_v, si0, si1, sg0, sg1):
        idxs, brys, rows = (idx0, idx1), (bry0, bry1), (rows0, rows1)
        sin, sg = (si0, si1), (sg0, sg1)
        wid = lax.axis_index("c") * _NS + lax.axis_index("s")
        iota = lax.iota(jnp.int32, _L)
        iota3 = iota * 3
        iota9 = iota * 9
        tec_base = wid * (nchunk * _PCB)
        pltpu.sync_copy(cam_hbm, cam_v)
        outs = (o0_v, o1_v, o2_v)
        out_hbms = (o0_hbm, o1_hbm, o2_hbm)

        def start_in(ch, b):
            pbase = tec_base + ch * _PCB
            pltpu.async_copy(p2f_hbm.at[pl.ds(pbase * 3, _PCB * 3)],
                             idxs[b], sin[b])
            pltpu.async_copy(bary_hbm.at[pl.ds(pbase * 9, _PCB * 9)],
                             brys[b], sin[b])

        def wait_in(b):
            pltpu.make_async_copy(p2f_hbm.at[pl.ds(0, _PCB * 3)],
                                  idxs[b], sin[b]).wait()
            pltpu.make_async_copy(bary_hbm.at[pl.ds(0, _PCB * 9)],
                                  brys[b], sin[b]).wait()

        def start_gather(b):
            pltpu.async_copy(ftab_hbm.at[idxs[b]], rows[b], sg[b])

        def wait_gather(b):
            pltpu.make_async_copy(ftab_hbm.at[idxs[b]], rows[b], sg[b]).wait()

        def do_chunk(ch, b, pred_next, pred_next2):
            q = 1 - b
            pbase = tec_base + ch * _PCB
            bidx = pbase // pix_per_batch
            zero16 = jnp.zeros((_L,), jnp.int32)
            cam = [plsc.load_gather(cam_v, [zero16 + (bidx * 3 + m)])
                   for m in range(3)]
            wait_gather(b)

            def _next():
                wait_in(q)
                start_gather(q)
            _maybe(pred_next, _next)

            def group_body(g, c2):
                for k in range(3):
                    row = g * 48 + iota3 + k
                    vv = [[plsc.load_gather(rows[b], [row, _col(j * 4 + m)])
                           for m in range(3)] for j in range(3)]
                    nn = [plsc.load_gather(rows[b], [row, _col(12 + m)])
                          for m in range(3)]
                    bbase = g * 144 + iota9 + k * 3
                    bb = [plsc.load_gather(brys[b], [bbase + j])
                          for j in range(3)]
                    pts = [bb[0] * vv[0][m] + bb[1] * vv[1][m]
                           + bb[2] * vv[2][m] for m in range(3)]
                    view = [pts[m] - cam[m] for m in range(3)]
                    len2 = jnp.maximum(view[0] * view[0] + view[1] * view[1]
                                       + view[2] * view[2], 1e-24)
                    r = _rsqrt(len2)
                    d = (nn[0] * view[0] + nn[1] * view[1]
                         + nn[2] * view[2]) * r
                    outs[k][pl.ds(g * 16, 16)] = d
                return c2

            lax.fori_loop(0, _PCB // _L, group_body, 0)
            for k in range(3):
                pltpu.sync_copy(outs[k], out_hbms[k].at[pl.ds(pbase, _PCB)])
            _maybe(pred_next2, lambda: start_in(ch + 2, b))

        start_in(0, 0)
        wait_in(0)
        start_gather(0)
        start_in(1, 1)

        def pair_body(cp, carry):
            for b in (0, 1):
                ch = cp * 2 + b
                do_chunk(ch, b, ch + 1 < nchunk, ch + 2 < nchunk)
            return carry

        lax.fori_loop(0, nchunk // 2, pair_body, 0)
        if nchunk % 2:
            do_chunk(nchunk - 1, (nchunk - 1) % 2, False, False)

    return shade_kernel


def kernel(pix_to_face, bary_coords, verts, faces, cam_origin):
    n, h, w, k = pix_to_face.shape
    np_pix = n * h * w
    v_cnt = verts.shape[0]
    f_cnt = faces.shape[0]
    align = _NW * _FCB
    f_pad = ((f_cnt + align - 1) // align) * align

    verts_pad = jnp.zeros((v_cnt, 16), jnp.float32).at[:, :3].set(verts)
    faces_flat = jnp.concatenate(
        [faces.reshape(-1),
         jnp.zeros((f_pad - f_cnt) * 3, jnp.int32)])
    p2f_flat = pix_to_face.reshape(-1)
    bary_flat = bary_coords.reshape(-1)
    cam_pad = jnp.zeros((16,), jnp.float32).at[: n * 3].set(
        cam_origin.reshape(-1))

    ftab = _make_facetab_kernel(f_pad)(verts_pad, faces_flat)
    o0, o1, o2 = _make_shade_kernel(np_pix, f_pad, h * w)(
        ftab.reshape(f_pad, 16), p2f_flat, bary_flat, cam_pad)
    return tuple(o.reshape(n, h, w, 1) for o in (o0, o1, o2))


# speedup vs baseline: 107.5409x; 5.1305x over previous
import functools

import jax
import jax.numpy as jnp
from jax import lax
from jax.experimental import pallas as pl
from jax.experimental.pallas import tpu as pltpu
from jax.experimental.pallas import tpu_sc as plsc

_NC = 2
_NS = 16
_NW = _NC * _NS
_L = 16

_FCB = 256
_PCB = 256

_PARAMS = pltpu.CompilerParams(needs_layout_passes=False,
                               use_tc_tiling_on_sc=False)


def _rsqrt(x):
    i = plsc.bitcast(x, jnp.int32)
    y = plsc.bitcast(jnp.int32(0x5F3759DF) - (i >> 1), jnp.float32)
    xh = x * 0.5
    for _ in range(3):
        y = y * (1.5 - xh * y * y)
    return y


def _col(c):
    return jnp.full((_L,), c, jnp.int32)


def _maybe(pred, fn):
    if isinstance(pred, bool):
        if pred:
            fn()
    else:
        pl.when(pred)(fn)


def _make_facetab_kernel(f_pad):
    nchunk = f_pad // (_NW * _FCB)
    mesh = plsc.VectorSubcoreMesh(core_axis_name="c", subcore_axis_name="s")

    @functools.partial(
        pl.kernel,
        out_type=jax.ShapeDtypeStruct((f_pad * 16,), jnp.float32),
        mesh=mesh,
        compiler_params=_PARAMS,
        scratch_types=[
            pltpu.VMEM((_FCB * 3,), jnp.int32),
            pltpu.VMEM((_FCB * 3,), jnp.int32),
            pltpu.VMEM((_FCB * 3, 16), jnp.float32),
            pltpu.VMEM((_FCB * 3, 16), jnp.float32),
            pltpu.VMEM((_FCB * 16,), jnp.float32),
            pltpu.SemaphoreType.DMA,
            pltpu.SemaphoreType.DMA,
            pltpu.SemaphoreType.DMA,
            pltpu.SemaphoreType.DMA,
        ],
    )
    def facetab_kernel(verts_hbm, faces_hbm, ftab_hbm, fidx0, fidx1, vrows0,
                       vrows1, fout_v, si0, si1, sg0, sg1):
        fidxs, vrows = (fidx0, fidx1), (vrows0, vrows1)
        sin, sg = (si0, si1), (sg0, sg1)
        wid = lax.axis_index("c") * _NS + lax.axis_index("s")
        iota = lax.iota(jnp.int32, _L)
        iota3 = iota * 3
        iota16 = iota * 16
        tec_base = wid * (nchunk * _FCB)

        def start_in(ch, b):
            fbase = tec_base + ch * _FCB
            pltpu.async_copy(faces_hbm.at[pl.ds(fbase * 3, _FCB * 3)],
                             fidxs[b], sin[b])

        def wait_in(b):
            pltpu.make_async_copy(faces_hbm.at[pl.ds(0, _FCB * 3)],
                                  fidxs[b], sin[b]).wait()

        def start_gather(b):
            pltpu.async_copy(verts_hbm.at[fidxs[b]], vrows[b], sg[b])

        def wait_gather(b):
            pltpu.make_async_copy(verts_hbm.at[fidxs[b]], vrows[b],
                                  sg[b]).wait()

        def do_chunk(ch, b, pred_next, pred_next2):
            q = 1 - b
            wait_gather(b)

            def _next():
                wait_in(q)
                start_gather(q)
            _maybe(pred_next, _next)

            def group_body(g, c2):
                v = []
                for j in range(3):
                    row = g * 48 + iota3 + j
                    v.append([plsc.load_gather(vrows[b], [row, _col(m)])
                              for m in range(3)])
                e1 = [v[1][m] - v[0][m] for m in range(3)]
                e2 = [v[2][m] - v[0][m] for m in range(3)]
                n = [e1[1] * e2[2] - e1[2] * e2[1],
                     e1[2] * e2[0] - e1[0] * e2[2],
                     e1[0] * e2[1] - e1[1] * e2[0]]
                len2 = jnp.maximum(n[0] * n[0] + n[1] * n[1] + n[2] * n[2],
                                   1e-24)
                r = _rsqrt(len2)
                obase = g * 256 + iota16
                for j in range(3):
                    for m in range(3):
                        plsc.store_scatter(fout_v, [obase + (j * 4 + m)],
                                           v[j][m])
                for m in range(3):
                    plsc.store_scatter(fout_v, [obase + (12 + m)], n[m] * r)
                return c2

            lax.fori_loop(0, _FCB // _L, group_body, 0)
            fbase = tec_base + ch * _FCB
            pltpu.sync_copy(fout_v, ftab_hbm.at[pl.ds(fbase * 16, _FCB * 16)])
            _maybe(pred_next2, lambda: start_in(ch + 2, b))

        start_in(0, 0)
        wait_in(0)
        start_gather(0)
        start_in(1, 1)

        def pair_body(cp, carry):
            for b in (0, 1):
                ch = cp * 2 + b
                do_chunk(ch, b, ch + 1 < nchunk, ch + 2 < nchunk)
            return carry

        lax.fori_loop(0, nchunk // 2, pair_body, 0)
        if nchunk % 2:
            do_chunk(nchunk - 1, (nchunk - 1) % 2, False, False)

    return facetab_kernel


def _make_shade_kernel(np_pix, f_pad, pix_per_batch):
    nchunk = np_pix // (_NW * _PCB)
    mesh = plsc.VectorSubcoreMesh(core_axis_name="c", subcore_axis_name="s")
    out = jax.ShapeDtypeStruct((np_pix,), jnp.float32)

    @functools.partial(
        pl.kernel,
        out_type=(out, out, out),
        mesh=mesh,
        compiler_params=_PARAMS,
        scratch_types=[
            pltpu.VMEM((_PCB * 3,), jnp.int32),
            pltpu.VMEM((_PCB * 3,), jnp.int32),
            pltpu.VMEM((_PCB * 9,), jnp.float32),
            pltpu.VMEM((_PCB * 9,), jnp.float32),
            pltpu.VMEM((_PCB * 3, 16), jnp.float32),
            pltpu.VMEM((_PCB * 3, 16), jnp.float32),
            pltpu.VMEM((_PCB,), jnp.float32),
            pltpu.VMEM((_PCB,), jnp.float32),
            pltpu.VMEM((_PCB,), jnp.float32),
            pltpu.VMEM((16,), jnp.float32),
            pltpu.SemaphoreType.DMA,
            pltpu.SemaphoreType.DMA,
            pltpu.SemaphoreType.DMA,
            pltpu.SemaphoreType.DMA,
        ],
    )
    def shade_kernel(ftab_hbm, p2f_hbm, bary_hbm, cam_hbm, o0_hbm, o1_hbm,
                     o2_hbm, idx0, idx1, bry0, bry1, rows0, rows1, o0_v, o1_v,
                     o2_v, cam_v, si0, si1, sg0, sg1):
        idxs, brys, rows = (idx0, idx1), (bry0, bry1), (rows0, rows1)
        sin, sg = (si0, si1), (sg0, sg1)
        wid = lax.axis_index("c") * _NS + lax.axis_index("s")
        iota = lax.iota(jnp.int32, _L)
        tec_base = wid * (nchunk * _PCB)
        pltpu.sync_copy(cam_hbm, cam_v)
        outs = (o0_v, o1_v, o2_v)
        out_hbms = (o0_hbm, o1_hbm, o2_hbm)

        def start_in(ch, b):
            pbase = tec_base + ch * _PCB
            n = pbase // pix_per_batch
            pp = pbase - n * pix_per_batch
            for k in range(3):
                pltpu.async_copy(
                    p2f_hbm.at[pl.ds((n * 3 + k) * pix_per_batch + pp, _PCB)],
                    idxs[b].at[pl.ds(k * _PCB, _PCB)], sin[b])
            for kc in range(9):
                pltpu.async_copy(
                    bary_hbm.at[pl.ds((n * 9 + kc) * pix_per_batch + pp,
                                      _PCB)],
                    brys[b].at[pl.ds(kc * _PCB, _PCB)], sin[b])

        def wait_in(b):
            pltpu.make_async_copy(p2f_hbm.at[pl.ds(0, _PCB * 3)],
                                  idxs[b], sin[b]).wait()
            pltpu.make_async_copy(bary_hbm.at[pl.ds(0, _PCB * 9)],
                                  brys[b], sin[b]).wait()

        def start_gather(b):
            pltpu.async_copy(ftab_hbm.at[idxs[b]], rows[b], sg[b])

        def wait_gather(b):
            pltpu.make_async_copy(ftab_hbm.at[idxs[b]], rows[b], sg[b]).wait()

        def do_chunk(ch, b, pred_next, pred_next2):
            q = 1 - b
            pbase = tec_base + ch * _PCB
            bidx = pbase // pix_per_batch
            zero16 = jnp.zeros((_L,), jnp.int32)
            cam = [plsc.load_gather(cam_v, [zero16 + (bidx * 3 + m)])
                   for m in range(3)]
            wait_gather(b)

            def _next():
                wait_in(q)
                start_gather(q)
            _maybe(pred_next, _next)

            def group_body(g, c2):
                for k in range(3):
                    row = k * _PCB + g * 16 + iota
                    vv = [[plsc.load_gather(rows[b], [row, _col(j * 4 + m)])
                           for m in range(3)] for j in range(3)]
                    nn = [plsc.load_gather(rows[b], [row, _col(12 + m)])
                          for m in range(3)]
                    bb = [brys[b][pl.ds((k * 3 + j) * _PCB + g * 16, 16)]
                          for j in range(3)]
                    pts = [bb[0] * vv[0][m] + bb[1] * vv[1][m]
                           + bb[2] * vv[2][m] for m in range(3)]
                    view = [pts[m] - cam[m] for m in range(3)]
                    len2 = jnp.maximum(view[0] * view[0] + view[1] * view[1]
                                       + view[2] * view[2], 1e-24)
                    r = _rsqrt(len2)
                    d = (nn[0] * view[0] + nn[1] * view[1]
                         + nn[2] * view[2]) * r
                    outs[k][pl.ds(g * 16, 16)] = d
                return c2

            lax.fori_loop(0, _PCB // _L, group_body, 0)
            for k in range(3):
                pltpu.sync_copy(outs[k], out_hbms[k].at[pl.ds(pbase, _PCB)])
            _maybe(pred_next2, lambda: start_in(ch + 2, b))

        start_in(0, 0)
        wait_in(0)
        start_gather(0)
        start_in(1, 1)

        def pair_body(cp, carry):
            for b in (0, 1):
                ch = cp * 2 + b
                do_chunk(ch, b, ch + 1 < nchunk, ch + 2 < nchunk)
            return carry

        lax.fori_loop(0, nchunk // 2, pair_body, 0)
        if nchunk % 2:
            do_chunk(nchunk - 1, (nchunk - 1) % 2, False, False)

    return shade_kernel


def kernel(pix_to_face, bary_coords, verts, faces, cam_origin):
    n, h, w, k = pix_to_face.shape
    np_pix = n * h * w
    v_cnt = verts.shape[0]
    f_cnt = faces.shape[0]
    align = _NW * _FCB
    f_pad = ((f_cnt + align - 1) // align) * align

    verts_pad = jnp.zeros((v_cnt, 16), jnp.float32).at[:, :3].set(verts)
    faces_flat = jnp.concatenate(
        [faces.reshape(-1),
         jnp.zeros((f_pad - f_cnt) * 3, jnp.int32)])
    p2f_flat = pix_to_face.transpose(0, 3, 1, 2).reshape(-1)
    bary_flat = bary_coords.transpose(0, 3, 4, 1, 2).reshape(-1)
    cam_pad = jnp.zeros((16,), jnp.float32).at[: n * 3].set(
        cam_origin.reshape(-1))

    ftab = _make_facetab_kernel(f_pad)(verts_pad, faces_flat)
    o0, o1, o2 = _make_shade_kernel(np_pix, f_pad, h * w)(
        ftab.reshape(f_pad, 16), p2f_flat, bary_flat, cam_pad)
    return tuple(o.reshape(n, h, w, 1) for o in (o0, o1, o2))


# speedup vs baseline: 113.7766x; 1.0580x over previous
import functools

import jax
import jax.numpy as jnp
from jax import lax
from jax.experimental import pallas as pl
from jax.experimental.pallas import tpu as pltpu
from jax.experimental.pallas import tpu_sc as plsc

_NC = 2
_NS = 16
_NW = _NC * _NS
_L = 16

_FCB = 256
_PCB = 512

_PARAMS = pltpu.CompilerParams(needs_layout_passes=False,
                               use_tc_tiling_on_sc=False)


def _rsqrt(x, steps=3):
    i = plsc.bitcast(x, jnp.int32)
    y = plsc.bitcast(jnp.int32(0x5F3759DF) - (i >> 1), jnp.float32)
    xh = x * 0.5
    for _ in range(steps):
        y = y * (1.5 - xh * y * y)
    return y


def _col(c):
    return jnp.full((_L,), c, jnp.int32)


def _maybe(pred, fn):
    if isinstance(pred, bool):
        if pred:
            fn()
    else:
        pl.when(pred)(fn)


def _make_facetab_kernel(f_pad):
    nchunk = f_pad // (_NW * _FCB)
    mesh = plsc.VectorSubcoreMesh(core_axis_name="c", subcore_axis_name="s")

    @functools.partial(
        pl.kernel,
        out_type=jax.ShapeDtypeStruct((f_pad * 16,), jnp.float32),
        mesh=mesh,
        compiler_params=_PARAMS,
        scratch_types=[
            pltpu.VMEM((_FCB * 3,), jnp.int32),
            pltpu.VMEM((_FCB * 3,), jnp.int32),
            pltpu.VMEM((_FCB * 3, 16), jnp.float32),
            pltpu.VMEM((_FCB * 3, 16), jnp.float32),
            pltpu.VMEM((_FCB * 16,), jnp.float32),
            pltpu.SemaphoreType.DMA,
            pltpu.SemaphoreType.DMA,
            pltpu.SemaphoreType.DMA,
            pltpu.SemaphoreType.DMA,
        ],
    )
    def facetab_kernel(verts_hbm, faces_hbm, ftab_hbm, fidx0, fidx1, vrows0,
                       vrows1, fout_v, si0, si1, sg0, sg1):
        fidxs, vrows = (fidx0, fidx1), (vrows0, vrows1)
        sin, sg = (si0, si1), (sg0, sg1)
        wid = lax.axis_index("c") * _NS + lax.axis_index("s")
        iota = lax.iota(jnp.int32, _L)
        iota3 = iota * 3
        iota16 = iota * 16
        tec_base = wid * (nchunk * _FCB)

        def start_in(ch, b):
            fbase = tec_base + ch * _FCB
            pltpu.async_copy(faces_hbm.at[pl.ds(fbase * 3, _FCB * 3)],
                             fidxs[b], sin[b])

        def wait_in(b):
            pltpu.make_async_copy(faces_hbm.at[pl.ds(0, _FCB * 3)],
                                  fidxs[b], sin[b]).wait()

        def start_gather(b):
            pltpu.async_copy(verts_hbm.at[fidxs[b]], vrows[b], sg[b])

        def wait_gather(b):
            pltpu.make_async_copy(verts_hbm.at[fidxs[b]], vrows[b],
                                  sg[b]).wait()

        def do_chunk(ch, b, pred_next, pred_next2):
            q = 1 - b
            wait_gather(b)

            def _next():
                wait_in(q)
                start_gather(q)
            _maybe(pred_next, _next)

            def group_body(g, c2):
                v = []
                for j in range(3):
                    row = g * 48 + iota3 + j
                    v.append([plsc.load_gather(vrows[b], [row, _col(m)])
                              for m in range(3)])
                e1 = [v[1][m] - v[0][m] for m in range(3)]
                e2 = [v[2][m] - v[0][m] for m in range(3)]
                n = [e1[1] * e2[2] - e1[2] * e2[1],
                     e1[2] * e2[0] - e1[0] * e2[2],
                     e1[0] * e2[1] - e1[1] * e2[0]]
                len2 = jnp.maximum(n[0] * n[0] + n[1] * n[1] + n[2] * n[2],
                                   1e-24)
                r = _rsqrt(len2)
                obase = g * 256 + iota16
                for j in range(3):
                    for m in range(3):
                        plsc.store_scatter(fout_v, [obase + (j * 4 + m)],
                                           v[j][m])
                for m in range(3):
                    plsc.store_scatter(fout_v, [obase + (12 + m)], n[m] * r)
                return c2

            lax.fori_loop(0, _FCB // _L, group_body, 0)
            fbase = tec_base + ch * _FCB
            pltpu.sync_copy(fout_v, ftab_hbm.at[pl.ds(fbase * 16, _FCB * 16)])
            _maybe(pred_next2, lambda: start_in(ch + 2, b))

        start_in(0, 0)
        wait_in(0)
        start_gather(0)
        start_in(1, 1)

        def pair_body(cp, carry):
            for b in (0, 1):
                ch = cp * 2 + b
                do_chunk(ch, b, ch + 1 < nchunk, ch + 2 < nchunk)
            return carry

        lax.fori_loop(0, nchunk // 2, pair_body, 0)
        if nchunk % 2:
            do_chunk(nchunk - 1, (nchunk - 1) % 2, False, False)

    return facetab_kernel


def _make_shade_kernel(np_pix, f_pad, pix_per_batch):
    nchunk = np_pix // (_NW * _PCB)
    mesh = plsc.VectorSubcoreMesh(core_axis_name="c", subcore_axis_name="s")
    out = jax.ShapeDtypeStruct((np_pix,), jnp.float32)

    @functools.partial(
        pl.kernel,
        out_type=(out, out, out),
        mesh=mesh,
        compiler_params=_PARAMS,
        scratch_types=[
            pltpu.VMEM((_PCB * 3,), jnp.int32),
            pltpu.VMEM((_PCB * 3,), jnp.int32),
            pltpu.VMEM((_PCB * 9,), jnp.float32),
            pltpu.VMEM((_PCB * 9,), jnp.float32),
            pltpu.VMEM((_PCB * 3, 16), jnp.float32),
            pltpu.VMEM((_PCB * 3, 16), jnp.float32),
            pltpu.VMEM((_PCB,), jnp.float32),
            pltpu.VMEM((_PCB,), jnp.float32),
            pltpu.VMEM((_PCB,), jnp.float32),
            pltpu.VMEM((16,), jnp.float32),
            pltpu.SemaphoreType.DMA,
            pltpu.SemaphoreType.DMA,
            pltpu.SemaphoreType.DMA,
            pltpu.SemaphoreType.DMA,
        ],
    )
    def shade_kernel(ftab_hbm, p2f_hbm, bary_hbm, cam_hbm, o0_hbm, o1_hbm,
                     o2_hbm, idx0, idx1, bry0, bry1, rows0, rows1, o0_v, o1_v,
                     o2_v, cam_v, si0, si1, sg0, sg1):
        idxs, brys, rows = (idx0, idx1), (bry0, bry1), (rows0, rows1)
        sin, sg = (si0, si1), (sg0, sg1)
        wid = lax.axis_index("c") * _NS + lax.axis_index("s")
        iota = lax.iota(jnp.int32, _L)
        tec_base = wid * (nchunk * _PCB)
        pltpu.sync_copy(cam_hbm, cam_v)
        outs = (o0_v, o1_v, o2_v)
        out_hbms = (o0_hbm, o1_hbm, o2_hbm)

        def start_in(ch, b):
            pbase = tec_base + ch * _PCB
            n = pbase // pix_per_batch
            pp = pbase - n * pix_per_batch
            for k in range(3):
                pltpu.async_copy(
                    p2f_hbm.at[pl.ds((n * 3 + k) * pix_per_batch + pp, _PCB)],
                    idxs[b].at[pl.ds(k * _PCB, _PCB)], sin[b])
            for kc in range(9):
                pltpu.async_copy(
                    bary_hbm.at[pl.ds((n * 9 + kc) * pix_per_batch + pp,
                                      _PCB)],
                    brys[b].at[pl.ds(kc * _PCB, _PCB)], sin[b])

        def wait_in(b):
            pltpu.make_async_copy(p2f_hbm.at[pl.ds(0, _PCB * 3)],
                                  idxs[b], sin[b]).wait()
            pltpu.make_async_copy(bary_hbm.at[pl.ds(0, _PCB * 9)],
                                  brys[b], sin[b]).wait()

        def start_gather(b):
            pltpu.async_copy(ftab_hbm.at[idxs[b]], rows[b], sg[b])

        def wait_gather(b):
            pltpu.make_async_copy(ftab_hbm.at[idxs[b]], rows[b], sg[b]).wait()

        def do_chunk(ch, b, pred_next, pred_next2):
            q = 1 - b
            pbase = tec_base + ch * _PCB
            bidx = pbase // pix_per_batch
            zero16 = jnp.zeros((_L,), jnp.int32)
            cam = [plsc.load_gather(cam_v, [zero16 + (bidx * 3 + m)])
                   for m in range(3)]
            wait_gather(b)

            def _next():
                wait_in(q)
                start_gather(q)
            _maybe(pred_next, _next)

            def group_body(g, c2):
                for k in range(3):
                    row = k * _PCB + g * 16 + iota
                    vv = [[plsc.load_gather(rows[b], [row, _col(j * 4 + m)])
                           for m in range(3)] for j in range(3)]
                    nn = [plsc.load_gather(rows[b], [row, _col(12 + m)])
                          for m in range(3)]
                    bb = [brys[b][pl.ds((k * 3 + j) * _PCB + g * 16, 16)]
                          for j in range(3)]
                    pts = [bb[0] * vv[0][m] + bb[1] * vv[1][m]
                           + bb[2] * vv[2][m] for m in range(3)]
                    view = [pts[m] - cam[m] for m in range(3)]
                    len2 = jnp.maximum(view[0] * view[0] + view[1] * view[1]
                                       + view[2] * view[2], 1e-24)
                    r = _rsqrt(len2, steps=2)
                    d = (nn[0] * view[0] + nn[1] * view[1]
                         + nn[2] * view[2]) * r
                    outs[k][pl.ds(g * 16, 16)] = d
                return c2

            lax.fori_loop(0, _PCB // _L, group_body, 0)
            for k in range(3):
                pltpu.sync_copy(outs[k], out_hbms[k].at[pl.ds(pbase, _PCB)])
            _maybe(pred_next2, lambda: start_in(ch + 2, b))

        start_in(0, 0)
        wait_in(0)
        start_gather(0)
        start_in(1, 1)

        def pair_body(cp, carry):
            for b in (0, 1):
                ch = cp * 2 + b
                do_chunk(ch, b, ch + 1 < nchunk, ch + 2 < nchunk)
            return carry

        lax.fori_loop(0, nchunk // 2, pair_body, 0)
        if nchunk % 2:
            do_chunk(nchunk - 1, (nchunk - 1) % 2, False, False)

    return shade_kernel


def kernel(pix_to_face, bary_coords, verts, faces, cam_origin):
    n, h, w, k = pix_to_face.shape
    np_pix = n * h * w
    v_cnt = verts.shape[0]
    f_cnt = faces.shape[0]
    align = _NW * _FCB
    f_pad = ((f_cnt + align - 1) // align) * align

    verts_pad = jnp.zeros((v_cnt, 16), jnp.float32).at[:, :3].set(verts)
    faces_flat = jnp.concatenate(
        [faces.reshape(-1),
         jnp.zeros((f_pad - f_cnt) * 3, jnp.int32)])
    p2f_flat = pix_to_face.transpose(0, 3, 1, 2).reshape(-1)
    bary_flat = bary_coords.transpose(0, 3, 4, 1, 2).reshape(-1)
    cam_pad = jnp.zeros((16,), jnp.float32).at[: n * 3].set(
        cam_origin.reshape(-1))

    ftab = _make_facetab_kernel(f_pad)(verts_pad, faces_flat)
    o0, o1, o2 = _make_shade_kernel(np_pix, f_pad, h * w)(
        ftab.reshape(f_pad, 16), p2f_flat, bary_flat, cam_pad)
    return tuple(o.reshape(n, h, w, 1) for o in (o0, o1, o2))


# speedup vs baseline: 119.8236x; 1.0531x over previous
import functools

import jax
import jax.numpy as jnp
from jax import lax
from jax.experimental import pallas as pl
from jax.experimental.pallas import tpu as pltpu
from jax.experimental.pallas import tpu_sc as plsc

_NC = 2
_NS = 16
_NW = _NC * _NS
_L = 16

_FCB = 256
_PCB = 512

_PARAMS = pltpu.CompilerParams(needs_layout_passes=False,
                               use_tc_tiling_on_sc=False)


def _rsqrt(x, steps=3):
    i = plsc.bitcast(x, jnp.int32)
    y = plsc.bitcast(jnp.int32(0x5F3759DF) - (i >> 1), jnp.float32)
    xh = x * 0.5
    for _ in range(steps):
        y = y * (1.5 - xh * y * y)
    return y


def _col(c):
    return jnp.full((_L,), c, jnp.int32)


def _maybe(pred, fn):
    if isinstance(pred, bool):
        if pred:
            fn()
    else:
        pl.when(pred)(fn)


def _make_facetab_kernel(f_pad):
    nchunk = f_pad // (_NW * _FCB)
    mesh = plsc.VectorSubcoreMesh(core_axis_name="c", subcore_axis_name="s")

    @functools.partial(
        pl.kernel,
        out_type=jax.ShapeDtypeStruct((f_pad * 16,), jnp.float32),
        mesh=mesh,
        compiler_params=_PARAMS,
        scratch_types=[
            pltpu.VMEM((_FCB * 3,), jnp.int32),
            pltpu.VMEM((_FCB * 3,), jnp.int32),
            pltpu.VMEM((_FCB * 3, 8), jnp.float32),
            pltpu.VMEM((_FCB * 3, 8), jnp.float32),
            pltpu.VMEM((_FCB * 16,), jnp.float32),
            pltpu.SemaphoreType.DMA,
            pltpu.SemaphoreType.DMA,
            pltpu.SemaphoreType.DMA,
            pltpu.SemaphoreType.DMA,
        ],
    )
    def facetab_kernel(verts_hbm, faces_hbm, ftab_hbm, fidx0, fidx1, vrows0,
                       vrows1, fout_v, si0, si1, sg0, sg1):
        fidxs, vrows = (fidx0, fidx1), (vrows0, vrows1)
        sin, sg = (si0, si1), (sg0, sg1)
        wid = lax.axis_index("c") * _NS + lax.axis_index("s")
        iota = lax.iota(jnp.int32, _L)
        iota3 = iota * 3
        iota16 = iota * 16
        tec_base = wid * (nchunk * _FCB)

        def start_in(ch, b):
            fbase = tec_base + ch * _FCB
            pltpu.async_copy(faces_hbm.at[pl.ds(fbase * 3, _FCB * 3)],
                             fidxs[b], sin[b])

        def wait_in(b):
            pltpu.make_async_copy(faces_hbm.at[pl.ds(0, _FCB * 3)],
                                  fidxs[b], sin[b]).wait()

        def start_gather(b):
            pltpu.async_copy(verts_hbm.at[fidxs[b]], vrows[b], sg[b])

        def wait_gather(b):
            pltpu.make_async_copy(verts_hbm.at[fidxs[b]], vrows[b],
                                  sg[b]).wait()

        def do_chunk(ch, b, pred_next, pred_next2):
            q = 1 - b
            wait_gather(b)

            def _next():
                wait_in(q)
                start_gather(q)
            _maybe(pred_next, _next)

            def group_body(g, c2):
                v = []
                for j in range(3):
                    row = g * 48 + iota3 + j
                    v.append([plsc.load_gather(vrows[b], [row, _col(m)])
                              for m in range(3)])
                e1 = [v[1][m] - v[0][m] for m in range(3)]
                e2 = [v[2][m] - v[0][m] for m in range(3)]
                n = [e1[1] * e2[2] - e1[2] * e2[1],
                     e1[2] * e2[0] - e1[0] * e2[2],
                     e1[0] * e2[1] - e1[1] * e2[0]]
                len2 = jnp.maximum(n[0] * n[0] + n[1] * n[1] + n[2] * n[2],
                                   1e-24)
                r = _rsqrt(len2)
                obase = g * 256 + iota16
                for j in range(3):
                    for m in range(3):
                        plsc.store_scatter(fout_v, [obase + (j * 4 + m)],
                                           v[j][m])
                for m in range(3):
                    plsc.store_scatter(fout_v, [obase + (12 + m)], n[m] * r)
                return c2

            lax.fori_loop(0, _FCB // _L, group_body, 0)
            fbase = tec_base + ch * _FCB
            pltpu.sync_copy(fout_v, ftab_hbm.at[pl.ds(fbase * 16, _FCB * 16)])
            _maybe(pred_next2, lambda: start_in(ch + 2, b))

        start_in(0, 0)
        wait_in(0)
        start_gather(0)
        start_in(1, 1)

        def pair_body(cp, carry):
            for b in (0, 1):
                ch = cp * 2 + b
                do_chunk(ch, b, ch + 1 < nchunk, ch + 2 < nchunk)
            return carry

        lax.fori_loop(0, nchunk // 2, pair_body, 0)
        if nchunk % 2:
            do_chunk(nchunk - 1, (nchunk - 1) % 2, False, False)

    return facetab_kernel


def _make_shade_kernel(np_pix, f_pad, pix_per_batch):
    nchunk = np_pix // (_NW * _PCB)
    mesh = plsc.VectorSubcoreMesh(core_axis_name="c", subcore_axis_name="s")
    out = jax.ShapeDtypeStruct((np_pix,), jnp.float32)

    @functools.partial(
        pl.kernel,
        out_type=(out, out, out),
        mesh=mesh,
        compiler_params=_PARAMS,
        scratch_types=[
            pltpu.VMEM((_PCB * 3,), jnp.int32),
            pltpu.VMEM((_PCB * 3,), jnp.int32),
            pltpu.VMEM((_PCB * 9,), jnp.float32),
            pltpu.VMEM((_PCB * 9,), jnp.float32),
            pltpu.VMEM((_PCB * 3, 16), jnp.float32),
            pltpu.VMEM((_PCB * 3, 16), jnp.float32),
            pltpu.VMEM((_PCB,), jnp.float32),
            pltpu.VMEM((_PCB,), jnp.float32),
            pltpu.VMEM((_PCB,), jnp.float32),
            pltpu.VMEM((16,), jnp.float32),
            pltpu.SemaphoreType.DMA,
            pltpu.SemaphoreType.DMA,
            pltpu.SemaphoreType.DMA,
            pltpu.SemaphoreType.DMA,
        ],
    )
    def shade_kernel(ftab_hbm, p2f_hbm, bary_hbm, cam_hbm, o0_hbm, o1_hbm,
                     o2_hbm, idx0, idx1, bry0, bry1, rows0, rows1, o0_v, o1_v,
                     o2_v, cam_v, si0, si1, sg0, sg1):
        idxs, brys, rows = (idx0, idx1), (bry0, bry1), (rows0, rows1)
        sin, sg = (si0, si1), (sg0, sg1)
        wid = lax.axis_index("c") * _NS + lax.axis_index("s")
        iota = lax.iota(jnp.int32, _L)
        tec_base = wid * (nchunk * _PCB)
        pltpu.sync_copy(cam_hbm, cam_v)
        outs = (o0_v, o1_v, o2_v)
        out_hbms = (o0_hbm, o1_hbm, o2_hbm)

        def start_in(ch, b):
            pbase = tec_base + ch * _PCB
            n = pbase // pix_per_batch
            pp = pbase - n * pix_per_batch
            for k in range(3):
                pltpu.async_copy(
                    p2f_hbm.at[pl.ds((n * 3 + k) * pix_per_batch + pp, _PCB)],
                    idxs[b].at[pl.ds(k * _PCB, _PCB)], sin[b])
            for kc in range(9):
                pltpu.async_copy(
                    bary_hbm.at[pl.ds((n * 9 + kc) * pix_per_batch + pp,
                                      _PCB)],
                    brys[b].at[pl.ds(kc * _PCB, _PCB)], sin[b])

        def wait_in(b):
            pltpu.make_async_copy(p2f_hbm.at[pl.ds(0, _PCB * 3)],
                                  idxs[b], sin[b]).wait()
            pltpu.make_async_copy(bary_hbm.at[pl.ds(0, _PCB * 9)],
                                  brys[b], sin[b]).wait()

        half = (_PCB * 3) // 2

        def start_gather(b):
            pltpu.async_copy(ftab_hbm.at[idxs[b].at[pl.ds(0, half)]],
                             rows[b].at[pl.ds(0, half)], sg[b])
            pltpu.async_copy(ftab_hbm.at[idxs[b].at[pl.ds(half, half)]],
                             rows[b].at[pl.ds(half, half)], sg[b])

        def wait_gather(b):
            pltpu.make_async_copy(ftab_hbm.at[idxs[b]], rows[b], sg[b]).wait()

        def do_chunk(ch, b, pred_next, pred_next2):
            q = 1 - b
            pbase = tec_base + ch * _PCB
            bidx = pbase // pix_per_batch
            zero16 = jnp.zeros((_L,), jnp.int32)
            cam = [plsc.load_gather(cam_v, [zero16 + (bidx * 3 + m)])
                   for m in range(3)]
            wait_gather(b)

            def _next():
                wait_in(q)
                start_gather(q)
            _maybe(pred_next, _next)

            def group_body(g, c2):
                for k in range(3):
                    row = k * _PCB + g * 16 + iota
                    vv = [[plsc.load_gather(rows[b], [row, _col(j * 4 + m)])
                           for m in range(3)] for j in range(3)]
                    nn = [plsc.load_gather(rows[b], [row, _col(12 + m)])
                          for m in range(3)]
                    bb = [brys[b][pl.ds((k * 3 + j) * _PCB + g * 16, 16)]
                          for j in range(3)]
                    pts = [bb[0] * vv[0][m] + bb[1] * vv[1][m]
                           + bb[2] * vv[2][m] for m in range(3)]
                    view = [pts[m] - cam[m] for m in range(3)]
                    len2 = jnp.maximum(view[0] * view[0] + view[1] * view[1]
                                       + view[2] * view[2], 1e-24)
                    r = _rsqrt(len2, steps=2)
                    d = (nn[0] * view[0] + nn[1] * view[1]
                         + nn[2] * view[2]) * r
                    outs[k][pl.ds(g * 16, 16)] = d
                return c2

            lax.fori_loop(0, _PCB // _L, group_body, 0)
            for k in range(3):
                pltpu.sync_copy(outs[k], out_hbms[k].at[pl.ds(pbase, _PCB)])
            _maybe(pred_next2, lambda: start_in(ch + 2, b))

        start_in(0, 0)
        wait_in(0)
        start_gather(0)
        start_in(1, 1)

        def pair_body(cp, carry):
            for b in (0, 1):
                ch = cp * 2 + b
                do_chunk(ch, b, ch + 1 < nchunk, ch + 2 < nchunk)
            return carry

        lax.fori_loop(0, nchunk // 2, pair_body, 0)
        if nchunk % 2:
            do_chunk(nchunk - 1, (nchunk - 1) % 2, False, False)

    return shade_kernel


def kernel(pix_to_face, bary_coords, verts, faces, cam_origin):
    n, h, w, k = pix_to_face.shape
    np_pix = n * h * w
    v_cnt = verts.shape[0]
    f_cnt = faces.shape[0]
    align = _NW * _FCB
    f_pad = ((f_cnt + align - 1) // align) * align

    verts_pad = jnp.zeros((v_cnt, 8), jnp.float32).at[:, :3].set(verts)
    faces_flat = jnp.concatenate(
        [faces.reshape(-1),
         jnp.zeros((f_pad - f_cnt) * 3, jnp.int32)])
    p2f_flat = pix_to_face.transpose(0, 3, 1, 2).reshape(-1)
    bary_flat = bary_coords.transpose(0, 3, 4, 1, 2).reshape(-1)
    cam_pad = jnp.zeros((16,), jnp.float32).at[: n * 3].set(
        cam_origin.reshape(-1))

    ftab = _make_facetab_kernel(f_pad)(verts_pad, faces_flat)
    o0, o1, o2 = _make_shade_kernel(np_pix, f_pad, h * w)(
        ftab.reshape(f_pad, 16), p2f_flat, bary_flat, cam_pad)
    return tuple(o.reshape(n, h, w, 1) for o in (o0, o1, o2))


# speedup vs baseline: 163.7868x; 1.3669x over previous
import functools

import jax
import jax.numpy as jnp
from jax import lax
from jax.experimental import pallas as pl
from jax.experimental.pallas import tpu as pltpu
from jax.experimental.pallas import tpu_sc as plsc

_NC = 2
_NS = 16
_NW = _NC * _NS
_L = 16

_FCB = 256
_PCB = 512

_PARAMS = pltpu.CompilerParams(needs_layout_passes=False,
                               use_tc_tiling_on_sc=False)


def _rsqrt(x, steps=3):
    i = plsc.bitcast(x, jnp.int32)
    y = plsc.bitcast(jnp.int32(0x5F3759DF) - (i >> 1), jnp.float32)
    xh = x * 0.5
    for _ in range(steps):
        y = y * (1.5 - xh * y * y)
    return y


def _col(c):
    return jnp.full((_L,), c, jnp.int32)


def _maybe(pred, fn):
    if isinstance(pred, bool):
        if pred:
            fn()
    else:
        pl.when(pred)(fn)


def _make_facetab_kernel(f_pad):
    nchunk = f_pad // (_NW * _FCB)
    mesh = plsc.VectorSubcoreMesh(core_axis_name="c", subcore_axis_name="s")

    @functools.partial(
        pl.kernel,
        out_type=jax.ShapeDtypeStruct((f_pad * 16,), jnp.float32),
        mesh=mesh,
        compiler_params=_PARAMS,
        scratch_types=[
            pltpu.VMEM((_FCB * 3,), jnp.int32),
            pltpu.VMEM((_FCB * 3,), jnp.int32),
            pltpu.VMEM((_FCB * 3, 8), jnp.float32),
            pltpu.VMEM((_FCB * 3, 8), jnp.float32),
            pltpu.VMEM((_FCB * 16,), jnp.float32),
            pltpu.SemaphoreType.DMA,
            pltpu.SemaphoreType.DMA,
            pltpu.SemaphoreType.DMA,
            pltpu.SemaphoreType.DMA,
        ],
    )
    def facetab_kernel(verts_hbm, faces_hbm, ftab_hbm, fidx0, fidx1, vrows0,
                       vrows1, fout_v, si0, si1, sg0, sg1):
        fidxs, vrows = (fidx0, fidx1), (vrows0, vrows1)
        sin, sg = (si0, si1), (sg0, sg1)
        wid = lax.axis_index("c") * _NS + lax.axis_index("s")
        iota = lax.iota(jnp.int32, _L)
        iota16 = iota * 16
        tec_base = wid * (nchunk * _FCB)

        def start_in(ch, b):
            fbase = tec_base + ch * _FCB
            for j in range(3):
                pltpu.async_copy(
                    faces_hbm.at[pl.ds(j * f_pad + fbase, _FCB)],
                    fidxs[b].at[pl.ds(j * _FCB, _FCB)], sin[b])

        def wait_in(b):
            pltpu.make_async_copy(faces_hbm.at[pl.ds(0, _FCB * 3)],
                                  fidxs[b], sin[b]).wait()

        halfa = (_FCB * 3) // 2

        def start_gather(b):
            pltpu.async_copy(verts_hbm.at[fidxs[b].at[pl.ds(0, halfa)]],
                             vrows[b].at[pl.ds(0, halfa)], sg[b])
            pltpu.async_copy(verts_hbm.at[fidxs[b].at[pl.ds(halfa, halfa)]],
                             vrows[b].at[pl.ds(halfa, halfa)], sg[b])

        def wait_gather(b):
            pltpu.make_async_copy(verts_hbm.at[fidxs[b]], vrows[b],
                                  sg[b]).wait()

        def do_chunk(ch, b, pred_next, pred_next2):
            q = 1 - b

            def _next():
                wait_in(q)
                start_gather(q)
            _maybe(pred_next, _next)
            wait_gather(b)

            def group_body(g, c2):
                v = []
                for j in range(3):
                    row = j * _FCB + g * 16 + iota
                    v.append([plsc.load_gather(vrows[b], [row, _col(m)])
                              for m in range(3)])
                e1 = [v[1][m] - v[0][m] for m in range(3)]
                e2 = [v[2][m] - v[0][m] for m in range(3)]
                n = [e1[1] * e2[2] - e1[2] * e2[1],
                     e1[2] * e2[0] - e1[0] * e2[2],
                     e1[0] * e2[1] - e1[1] * e2[0]]
                len2 = jnp.maximum(n[0] * n[0] + n[1] * n[1] + n[2] * n[2],
                                   1e-24)
                r = _rsqrt(len2)
                obase = g * 256 + iota16
                for j in range(3):
                    for m in range(3):
                        plsc.store_scatter(fout_v, [obase + (j * 4 + m)],
                                           v[j][m])
                for m in range(3):
                    plsc.store_scatter(fout_v, [obase + (12 + m)], n[m] * r)
                return c2

            lax.fori_loop(0, _FCB // _L, group_body, 0)
            fbase = tec_base + ch * _FCB
            pltpu.sync_copy(fout_v, ftab_hbm.at[pl.ds(fbase * 16, _FCB * 16)])
            _maybe(pred_next2, lambda: start_in(ch + 2, b))

        start_in(0, 0)
        wait_in(0)
        start_gather(0)
        start_in(1, 1)

        def pair_body(cp, carry):
            for b in (0, 1):
                ch = cp * 2 + b
                do_chunk(ch, b, ch + 1 < nchunk, ch + 2 < nchunk)
            return carry

        lax.fori_loop(0, nchunk // 2, pair_body, 0)
        if nchunk % 2:
            do_chunk(nchunk - 1, (nchunk - 1) % 2, False, False)

    return facetab_kernel


def _make_shade_kernel(np_pix, f_pad, pix_per_batch):
    nchunk = np_pix // (_NW * _PCB)
    mesh = plsc.VectorSubcoreMesh(core_axis_name="c", subcore_axis_name="s")
    out = jax.ShapeDtypeStruct((np_pix,), jnp.float32)

    @functools.partial(
        pl.kernel,
        out_type=(out, out, out),
        mesh=mesh,
        compiler_params=_PARAMS,
        scratch_types=[
            pltpu.VMEM((_PCB * 3,), jnp.int32),
            pltpu.VMEM((_PCB * 3,), jnp.int32),
            pltpu.VMEM((_PCB * 9,), jnp.float32),
            pltpu.VMEM((_PCB * 9,), jnp.float32),
            pltpu.VMEM((_PCB * 3, 16), jnp.float32),
            pltpu.VMEM((_PCB * 3, 16), jnp.float32),
            pltpu.VMEM((_PCB,), jnp.float32),
            pltpu.VMEM((_PCB,), jnp.float32),
            pltpu.VMEM((_PCB,), jnp.float32),
            pltpu.VMEM((16,), jnp.float32),
            pltpu.SemaphoreType.DMA,
            pltpu.SemaphoreType.DMA,
            pltpu.SemaphoreType.DMA,
            pltpu.SemaphoreType.DMA,
        ],
    )
    def shade_kernel(ftab_hbm, p2f_hbm, bary_hbm, cam_hbm, o0_hbm, o1_hbm,
                     o2_hbm, idx0, idx1, bry0, bry1, rows0, rows1, o0_v, o1_v,
                     o2_v, cam_v, si0, si1, sg0, sg1):
        idxs, brys, rows = (idx0, idx1), (bry0, bry1), (rows0, rows1)
        sin, sg = (si0, si1), (sg0, sg1)
        wid = lax.axis_index("c") * _NS + lax.axis_index("s")
        iota = lax.iota(jnp.int32, _L)
        tec_base = wid * (nchunk * _PCB)
        pltpu.sync_copy(cam_hbm, cam_v)
        outs = (o0_v, o1_v, o2_v)
        out_hbms = (o0_hbm, o1_hbm, o2_hbm)

        def start_in(ch, b):
            pbase = tec_base + ch * _PCB
            n = pbase // pix_per_batch
            pp = pbase - n * pix_per_batch
            for k in range(3):
                pltpu.async_copy(
                    p2f_hbm.at[pl.ds((n * 3 + k) * pix_per_batch + pp, _PCB)],
                    idxs[b].at[pl.ds(k * _PCB, _PCB)], sin[b])
            for kc in range(9):
                pltpu.async_copy(
                    bary_hbm.at[pl.ds((n * 9 + kc) * pix_per_batch + pp,
                                      _PCB)],
                    brys[b].at[pl.ds(kc * _PCB, _PCB)], sin[b])

        def wait_in(b):
            pltpu.make_async_copy(p2f_hbm.at[pl.ds(0, _PCB * 3)],
                                  idxs[b], sin[b]).wait()
            pltpu.make_async_copy(bary_hbm.at[pl.ds(0, _PCB * 9)],
                                  brys[b], sin[b]).wait()

        half = (_PCB * 3) // 2

        def start_gather(b):
            pltpu.async_copy(ftab_hbm.at[idxs[b].at[pl.ds(0, half)]],
                             rows[b].at[pl.ds(0, half)], sg[b])
            pltpu.async_copy(ftab_hbm.at[idxs[b].at[pl.ds(half, half)]],
                             rows[b].at[pl.ds(half, half)], sg[b])

        def wait_gather(b):
            pltpu.make_async_copy(ftab_hbm.at[idxs[b]], rows[b], sg[b]).wait()

        def do_chunk(ch, b, pred_next, pred_next2):
            q = 1 - b
            pbase = tec_base + ch * _PCB
            bidx = pbase // pix_per_batch
            zero16 = jnp.zeros((_L,), jnp.int32)
            cam = [plsc.load_gather(cam_v, [zero16 + (bidx * 3 + m)])
                   for m in range(3)]

            def _next():
                wait_in(q)
                start_gather(q)
            _maybe(pred_next, _next)
            wait_gather(b)

            def group_body(g, c2):
                for k in range(3):
                    row = k * _PCB + g * 16 + iota
                    vv = [[plsc.load_gather(rows[b], [row, _col(j * 4 + m)])
                           for m in range(3)] for j in range(3)]
                    nn = [plsc.load_gather(rows[b], [row, _col(12 + m)])
                          for m in range(3)]
                    bb = [brys[b][pl.ds((k * 3 + j) * _PCB + g * 16, 16)]
                          for j in range(3)]
                    pts = [bb[0] * vv[0][m] + bb[1] * vv[1][m]
                           + bb[2] * vv[2][m] for m in range(3)]
                    view = [pts[m] - cam[m] for m in range(3)]
                    len2 = jnp.maximum(view[0] * view[0] + view[1] * view[1]
                                       + view[2] * view[2], 1e-24)
                    r = _rsqrt(len2, steps=2)
                    d = (nn[0] * view[0] + nn[1] * view[1]
                         + nn[2] * view[2]) * r
                    outs[k][pl.ds(g * 16, 16)] = d
                return c2

            lax.fori_loop(0, _PCB // _L, group_body, 0)
            for k in range(3):
                pltpu.sync_copy(outs[k], out_hbms[k].at[pl.ds(pbase, _PCB)])
            _maybe(pred_next2, lambda: start_in(ch + 2, b))

        start_in(0, 0)
        wait_in(0)
        start_gather(0)
        start_in(1, 1)

        def pair_body(cp, carry):
            for b in (0, 1):
                ch = cp * 2 + b
                do_chunk(ch, b, ch + 1 < nchunk, ch + 2 < nchunk)
            return carry

        lax.fori_loop(0, nchunk // 2, pair_body, 0)
        if nchunk % 2:
            do_chunk(nchunk - 1, (nchunk - 1) % 2, False, False)

    return shade_kernel


def kernel(pix_to_face, bary_coords, verts, faces, cam_origin):
    n, h, w, k = pix_to_face.shape
    np_pix = n * h * w
    v_cnt = verts.shape[0]
    f_cnt = faces.shape[0]
    align = _NW * _FCB
    f_pad = ((f_cnt + align - 1) // align) * align

    verts_pad = jnp.pad(verts, ((0, 0), (0, 5)))
    faces_flat = jnp.pad(faces, ((0, f_pad - f_cnt), (0, 0))).T.reshape(-1)
    p2f_flat = pix_to_face.transpose(0, 3, 1, 2).reshape(-1)
    bary_flat = bary_coords.transpose(0, 3, 4, 1, 2).reshape(-1)
    cam_pad = jnp.zeros((16,), jnp.float32).at[: n * 3].set(
        cam_origin.reshape(-1))

    ftab = _make_facetab_kernel(f_pad)(verts_pad, faces_flat)
    o0, o1, o2 = _make_shade_kernel(np_pix, f_pad, h * w)(
        ftab.reshape(f_pad, 16), p2f_flat, bary_flat, cam_pad)
    return tuple(o.reshape(n, h, w, 1) for o in (o0, o1, o2))


# speedup vs baseline: 188.8366x; 1.1529x over previous
import functools

import jax
import jax.numpy as jnp
from jax import lax
from jax.experimental import pallas as pl
from jax.experimental.pallas import tpu as pltpu
from jax.experimental.pallas import tpu_sc as plsc

_NC = 2
_NS = 16
_NW = _NC * _NS
_L = 16

_FCB = 256
_PCB = 512

_PARAMS = pltpu.CompilerParams(needs_layout_passes=False,
                               use_tc_tiling_on_sc=False)


def _rsqrt(x, steps=3):
    i = plsc.bitcast(x, jnp.int32)
    y = plsc.bitcast(jnp.int32(0x5F3759DF) - (i >> 1), jnp.float32)
    xh = x * 0.5
    for _ in range(steps):
        y = y * (1.5 - xh * y * y)
    return y


def _col(c):
    return jnp.full((_L,), c, jnp.int32)


def _pack2(a, b):
    ia = plsc.bitcast(a, jnp.int32)
    ia = ia + 0x7FFF + ((ia >> 16) & 1)
    ib = plsc.bitcast(b, jnp.int32)
    ib = ib + 0x7FFF + ((ib >> 16) & 1)
    word = (ia & jnp.int32(-65536)) | ((ib >> 16) & 0xFFFF)
    return plsc.bitcast(word, jnp.float32)


def _unpack2(w):
    i = plsc.bitcast(w, jnp.int32)
    hi = plsc.bitcast(i & jnp.int32(-65536), jnp.float32)
    lo = plsc.bitcast(i << 16, jnp.float32)
    return hi, lo


def _maybe(pred, fn):
    if isinstance(pred, bool):
        if pred:
            fn()
    else:
        pl.when(pred)(fn)


def _make_facetab_kernel(f_pad):
    nchunk = f_pad // (_NW * _FCB)
    mesh = plsc.VectorSubcoreMesh(core_axis_name="c", subcore_axis_name="s")

    @functools.partial(
        pl.kernel,
        out_type=jax.ShapeDtypeStruct((f_pad * 8,), jnp.float32),
        mesh=mesh,
        compiler_params=_PARAMS,
        scratch_types=[
            pltpu.VMEM((_FCB * 3,), jnp.int32),
            pltpu.VMEM((_FCB * 3,), jnp.int32),
            pltpu.VMEM((_FCB * 3, 8), jnp.float32),
            pltpu.VMEM((_FCB * 3, 8), jnp.float32),
            pltpu.VMEM((_FCB * 8,), jnp.float32),
            pltpu.SemaphoreType.DMA,
            pltpu.SemaphoreType.DMA,
            pltpu.SemaphoreType.DMA,
            pltpu.SemaphoreType.DMA,
        ],
    )
    def facetab_kernel(verts_hbm, faces_hbm, ftab_hbm, fidx0, fidx1, vrows0,
                       vrows1, fout_v, si0, si1, sg0, sg1):
        fidxs, vrows = (fidx0, fidx1), (vrows0, vrows1)
        sin, sg = (si0, si1), (sg0, sg1)
        wid = lax.axis_index("c") * _NS + lax.axis_index("s")
        iota = lax.iota(jnp.int32, _L)
        iota8 = iota * 8
        tec_base = wid * (nchunk * _FCB)

        def start_in(ch, b):
            fbase = tec_base + ch * _FCB
            for j in range(3):
                pltpu.async_copy(
                    faces_hbm.at[pl.ds(j * f_pad + fbase, _FCB)],
                    fidxs[b].at[pl.ds(j * _FCB, _FCB)], sin[b])

        def wait_in(b):
            pltpu.make_async_copy(faces_hbm.at[pl.ds(0, _FCB * 3)],
                                  fidxs[b], sin[b]).wait()

        halfa = (_FCB * 3) // 2

        def start_gather(b):
            pltpu.async_copy(verts_hbm.at[fidxs[b].at[pl.ds(0, halfa)]],
                             vrows[b].at[pl.ds(0, halfa)], sg[b])
            pltpu.async_copy(verts_hbm.at[fidxs[b].at[pl.ds(halfa, halfa)]],
                             vrows[b].at[pl.ds(halfa, halfa)], sg[b])

        def wait_gather(b):
            pltpu.make_async_copy(verts_hbm.at[fidxs[b]], vrows[b],
                                  sg[b]).wait()

        def do_chunk(ch, b, pred_next, pred_next2):
            q = 1 - b

            def _next():
                wait_in(q)
                start_gather(q)
            _maybe(pred_next, _next)
            wait_gather(b)

            def group_body(g, c2):
                v = []
                for j in range(3):
                    row = j * _FCB + g * 16 + iota
                    v.append([plsc.load_gather(vrows[b], [row, _col(m)])
                              for m in range(3)])
                e1 = [v[1][m] - v[0][m] for m in range(3)]
                e2 = [v[2][m] - v[0][m] for m in range(3)]
                n = [e1[1] * e2[2] - e1[2] * e2[1],
                     e1[2] * e2[0] - e1[0] * e2[2],
                     e1[0] * e2[1] - e1[1] * e2[0]]
                len2 = jnp.maximum(n[0] * n[0] + n[1] * n[1] + n[2] * n[2],
                                   1e-24)
                r = _rsqrt(len2)
                nrm = [n[m] * r for m in range(3)]
                words = [_pack2(v[0][0], v[0][1]), _pack2(v[0][2], v[1][0]),
                         _pack2(v[1][1], v[1][2]), _pack2(v[2][0], v[2][1]),
                         _pack2(v[2][2], nrm[0]), _pack2(nrm[1], nrm[2])]
                obase = g * 128 + iota8
                for wi in range(6):
                    plsc.store_scatter(fout_v, [obase + wi], words[wi])
                return c2

            lax.fori_loop(0, _FCB // _L, group_body, 0)
            fbase = tec_base + ch * _FCB
            pltpu.sync_copy(fout_v, ftab_hbm.at[pl.ds(fbase * 8, _FCB * 8)])
            _maybe(pred_next2, lambda: start_in(ch + 2, b))

        start_in(0, 0)
        wait_in(0)
        start_gather(0)
        start_in(1, 1)

        def pair_body(cp, carry):
            for b in (0, 1):
                ch = cp * 2 + b
                do_chunk(ch, b, ch + 1 < nchunk, ch + 2 < nchunk)
            return carry

        lax.fori_loop(0, nchunk // 2, pair_body, 0)
        if nchunk % 2:
            do_chunk(nchunk - 1, (nchunk - 1) % 2, False, False)

    return facetab_kernel


def _make_shade_kernel(np_pix, f_pad, pix_per_batch):
    nchunk = np_pix // (_NW * _PCB)
    mesh = plsc.VectorSubcoreMesh(core_axis_name="c", subcore_axis_name="s")
    out = jax.ShapeDtypeStruct((np_pix,), jnp.float32)

    @functools.partial(
        pl.kernel,
        out_type=(out, out, out),
        mesh=mesh,
        compiler_params=_PARAMS,
        scratch_types=[
            pltpu.VMEM((_PCB * 3,), jnp.int32),
            pltpu.VMEM((_PCB * 3,), jnp.int32),
            pltpu.VMEM((_PCB * 9,), jnp.float32),
            pltpu.VMEM((_PCB * 9,), jnp.float32),
            pltpu.VMEM((_PCB * 3, 8), jnp.float32),
            pltpu.VMEM((_PCB * 3, 8), jnp.float32),
            pltpu.VMEM((_PCB,), jnp.float32),
            pltpu.VMEM((_PCB,), jnp.float32),
            pltpu.VMEM((_PCB,), jnp.float32),
            pltpu.VMEM((16,), jnp.float32),
            pltpu.SemaphoreType.DMA,
            pltpu.SemaphoreType.DMA,
            pltpu.SemaphoreType.DMA,
            pltpu.SemaphoreType.DMA,
        ],
    )
    def shade_kernel(ftab_hbm, p2f_hbm, bary_hbm, cam_hbm, o0_hbm, o1_hbm,
                     o2_hbm, idx0, idx1, bry0, bry1, rows0, rows1, o0_v, o1_v,
                     o2_v, cam_v, si0, si1, sg0, sg1):
        idxs, brys, rows = (idx0, idx1), (bry0, bry1), (rows0, rows1)
        sin, sg = (si0, si1), (sg0, sg1)
        wid = lax.axis_index("c") * _NS + lax.axis_index("s")
        iota = lax.iota(jnp.int32, _L)
        tec_base = wid * (nchunk * _PCB)
        pltpu.sync_copy(cam_hbm, cam_v)
        outs = (o0_v, o1_v, o2_v)
        out_hbms = (o0_hbm, o1_hbm, o2_hbm)

        def start_in(ch, b):
            pbase = tec_base + ch * _PCB
            n = pbase // pix_per_batch
            pp = pbase - n * pix_per_batch
            for k in range(3):
                pltpu.async_copy(
                    p2f_hbm.at[pl.ds((n * 3 + k) * pix_per_batch + pp, _PCB)],
                    idxs[b].at[pl.ds(k * _PCB, _PCB)], sin[b])
            for kc in range(9):
                pltpu.async_copy(
                    bary_hbm.at[pl.ds((n * 9 + kc) * pix_per_batch + pp,
                                      _PCB)],
                    brys[b].at[pl.ds(kc * _PCB, _PCB)], sin[b])

        def wait_in(b):
            pltpu.make_async_copy(p2f_hbm.at[pl.ds(0, _PCB * 3)],
                                  idxs[b], sin[b]).wait()
            pltpu.make_async_copy(bary_hbm.at[pl.ds(0, _PCB * 9)],
                                  brys[b], sin[b]).wait()

        half = (_PCB * 3) // 2

        def start_gather(b):
            pltpu.async_copy(ftab_hbm.at[idxs[b].at[pl.ds(0, half)]],
                             rows[b].at[pl.ds(0, half)], sg[b])
            pltpu.async_copy(ftab_hbm.at[idxs[b].at[pl.ds(half, half)]],
                             rows[b].at[pl.ds(half, half)], sg[b])

        def wait_gather(b):
            pltpu.make_async_copy(ftab_hbm.at[idxs[b]], rows[b], sg[b]).wait()

        def do_chunk(ch, b, pred_next, pred_next2):
            q = 1 - b
            pbase = tec_base + ch * _PCB
            bidx = pbase // pix_per_batch
            zero16 = jnp.zeros((_L,), jnp.int32)
            cam = [plsc.load_gather(cam_v, [zero16 + (bidx * 3 + m)])
                   for m in range(3)]

            def _next():
                wait_in(q)
                start_gather(q)
            _maybe(pred_next, _next)
            wait_gather(b)

            def group_body(g, c2):
                for k in range(3):
                    row = k * _PCB + g * 16 + iota
                    w = [plsc.load_gather(rows[b], [row, _col(wi)])
                         for wi in range(6)]
                    v0x, v0y = _unpack2(w[0])
                    v0z, v1x = _unpack2(w[1])
                    v1y, v1z = _unpack2(w[2])
                    v2x, v2y = _unpack2(w[3])
                    v2z, n0 = _unpack2(w[4])
                    n1, n2 = _unpack2(w[5])
                    vv = [[v0x, v0y, v0z], [v1x, v1y, v1z], [v2x, v2y, v2z]]
                    nn = [n0, n1, n2]
                    bb = [brys[b][pl.ds((k * 3 + j) * _PCB + g * 16, 16)]
                          for j in range(3)]
                    pts = [bb[0] * vv[0][m] + bb[1] * vv[1][m]
                           + bb[2] * vv[2][m] for m in range(3)]
                    view = [pts[m] - cam[m] for m in range(3)]
                    len2 = jnp.maximum(view[0] * view[0] + view[1] * view[1]
                                       + view[2] * view[2], 1e-24)
                    r = _rsqrt(len2, steps=2)
                    d = (nn[0] * view[0] + nn[1] * view[1]
                         + nn[2] * view[2]) * r
                    outs[k][pl.ds(g * 16, 16)] = d
                return c2

            lax.fori_loop(0, _PCB // _L, group_body, 0)
            for k in range(3):
                pltpu.sync_copy(outs[k], out_hbms[k].at[pl.ds(pbase, _PCB)])
            _maybe(pred_next2, lambda: start_in(ch + 2, b))

        start_in(0, 0)
        wait_in(0)
        start_gather(0)
        start_in(1, 1)

        def pair_body(cp, carry):
            for b in (0, 1):
                ch = cp * 2 + b
                do_chunk(ch, b, ch + 1 < nchunk, ch + 2 < nchunk)
            return carry

        lax.fori_loop(0, nchunk // 2, pair_body, 0)
        if nchunk % 2:
            do_chunk(nchunk - 1, (nchunk - 1) % 2, False, False)

    return shade_kernel


def kernel(pix_to_face, bary_coords, verts, faces, cam_origin):
    n, h, w, k = pix_to_face.shape
    np_pix = n * h * w
    v_cnt = verts.shape[0]
    f_cnt = faces.shape[0]
    align = _NW * _FCB
    f_pad = ((f_cnt + align - 1) // align) * align

    verts_pad = jnp.pad(verts, ((0, 0), (0, 5)))
    faces_flat = jnp.pad(faces, ((0, f_pad - f_cnt), (0, 0))).T.reshape(-1)
    p2f_flat = pix_to_face.transpose(0, 3, 1, 2).reshape(-1)
    bary_flat = bary_coords.transpose(0, 3, 4, 1, 2).reshape(-1)
    cam_pad = jnp.zeros((16,), jnp.float32).at[: n * 3].set(
        cam_origin.reshape(-1))

    ftab = _make_facetab_kernel(f_pad)(verts_pad, faces_flat)
    o0, o1, o2 = _make_shade_kernel(np_pix, f_pad, h * w)(
        ftab.reshape(f_pad, 8), p2f_flat, bary_flat, cam_pad)
    return tuple(o.reshape(n, h, w, 1) for o in (o0, o1, o2))


# speedup vs baseline: 195.1505x; 1.0334x over previous
import functools

import jax
import jax.numpy as jnp
from jax import lax
from jax.experimental import pallas as pl
from jax.experimental.pallas import tpu as pltpu
from jax.experimental.pallas import tpu_sc as plsc

_NC = 2
_NS = 16
_NW = _NC * _NS
_L = 16

_FCB = 256
_PCB = 1024

_PARAMS = pltpu.CompilerParams(needs_layout_passes=False,
                               use_tc_tiling_on_sc=False)


def _rsqrt(x, steps=3):
    i = plsc.bitcast(x, jnp.int32)
    y = plsc.bitcast(jnp.int32(0x5F3759DF) - (i >> 1), jnp.float32)
    xh = x * 0.5
    for _ in range(steps):
        y = y * (1.5 - xh * y * y)
    return y


def _col(c):
    return jnp.full((_L,), c, jnp.int32)


def _pack2(a, b):
    ia = plsc.bitcast(a, jnp.int32)
    ia = ia + 0x7FFF + ((ia >> 16) & 1)
    ib = plsc.bitcast(b, jnp.int32)
    ib = ib + 0x7FFF + ((ib >> 16) & 1)
    word = (ia & jnp.int32(-65536)) | ((ib >> 16) & 0xFFFF)
    return plsc.bitcast(word, jnp.float32)


def _unpack2(w):
    i = plsc.bitcast(w, jnp.int32)
    hi = plsc.bitcast(i & jnp.int32(-65536), jnp.float32)
    lo = plsc.bitcast(i << 16, jnp.float32)
    return hi, lo


def _maybe(pred, fn):
    if isinstance(pred, bool):
        if pred:
            fn()
    else:
        pl.when(pred)(fn)


def _make_facetab_kernel(f_pad):
    nchunk = f_pad // (_NW * _FCB)
    mesh = plsc.VectorSubcoreMesh(core_axis_name="c", subcore_axis_name="s")

    @functools.partial(
        pl.kernel,
        out_type=jax.ShapeDtypeStruct((f_pad * 8,), jnp.float32),
        mesh=mesh,
        compiler_params=_PARAMS,
        scratch_types=[
            pltpu.VMEM((_FCB * 3,), jnp.int32),
            pltpu.VMEM((_FCB * 3,), jnp.int32),
            pltpu.VMEM((_FCB * 3, 8), jnp.float32),
            pltpu.VMEM((_FCB * 3, 8), jnp.float32),
            pltpu.VMEM((_FCB * 8,), jnp.float32),
            pltpu.SemaphoreType.DMA,
            pltpu.SemaphoreType.DMA,
            pltpu.SemaphoreType.DMA,
            pltpu.SemaphoreType.DMA,
        ],
    )
    def facetab_kernel(verts_hbm, faces_hbm, ftab_hbm, fidx0, fidx1, vrows0,
                       vrows1, fout_v, si0, si1, sg0, sg1):
        fidxs, vrows = (fidx0, fidx1), (vrows0, vrows1)
        sin, sg = (si0, si1), (sg0, sg1)
        wid = lax.axis_index("c") * _NS + lax.axis_index("s")
        iota = lax.iota(jnp.int32, _L)
        iota8 = iota * 8
        tec_base = wid * (nchunk * _FCB)

        def start_in(ch, b):
            fbase = tec_base + ch * _FCB
            for j in range(3):
                pltpu.async_copy(
                    faces_hbm.at[pl.ds(j * f_pad + fbase, _FCB)],
                    fidxs[b].at[pl.ds(j * _FCB, _FCB)], sin[b])

        def wait_in(b):
            pltpu.make_async_copy(faces_hbm.at[pl.ds(0, _FCB * 3)],
                                  fidxs[b], sin[b]).wait()

        halfa = (_FCB * 3) // 2

        def start_gather(b):
            pltpu.async_copy(verts_hbm.at[fidxs[b].at[pl.ds(0, halfa)]],
                             vrows[b].at[pl.ds(0, halfa)], sg[b])
            pltpu.async_copy(verts_hbm.at[fidxs[b].at[pl.ds(halfa, halfa)]],
                             vrows[b].at[pl.ds(halfa, halfa)], sg[b])

        def wait_gather(b):
            pltpu.make_async_copy(verts_hbm.at[fidxs[b]], vrows[b],
                                  sg[b]).wait()

        def do_chunk(ch, b, pred_next, pred_next2):
            q = 1 - b

            def _next():
                wait_in(q)
                start_gather(q)
            _maybe(pred_next, _next)
            wait_gather(b)

            def group_body(g, c2):
                v = []
                for j in range(3):
                    row = j * _FCB + g * 16 + iota
                    v.append([plsc.load_gather(vrows[b], [row, _col(m)])
                              for m in range(3)])
                e1 = [v[1][m] - v[0][m] for m in range(3)]
                e2 = [v[2][m] - v[0][m] for m in range(3)]
                n = [e1[1] * e2[2] - e1[2] * e2[1],
                     e1[2] * e2[0] - e1[0] * e2[2],
                     e1[0] * e2[1] - e1[1] * e2[0]]
                len2 = jnp.maximum(n[0] * n[0] + n[1] * n[1] + n[2] * n[2],
                                   1e-24)
                r = _rsqrt(len2)
                nrm = [n[m] * r for m in range(3)]
                words = [_pack2(v[0][0], v[0][1]), _pack2(v[0][2], v[1][0]),
                         _pack2(v[1][1], v[1][2]), _pack2(v[2][0], v[2][1]),
                         _pack2(v[2][2], nrm[0]), _pack2(nrm[1], nrm[2])]
                obase = g * 128 + iota8
                for wi in range(6):
                    plsc.store_scatter(fout_v, [obase + wi], words[wi])
                return c2

            lax.fori_loop(0, _FCB // _L, group_body, 0)
            fbase = tec_base + ch * _FCB
            pltpu.sync_copy(fout_v, ftab_hbm.at[pl.ds(fbase * 8, _FCB * 8)])
            _maybe(pred_next2, lambda: start_in(ch + 2, b))

        start_in(0, 0)
        wait_in(0)
        start_gather(0)
        start_in(1, 1)

        def pair_body(cp, carry):
            for b in (0, 1):
                ch = cp * 2 + b
                do_chunk(ch, b, ch + 1 < nchunk, ch + 2 < nchunk)
            return carry

        lax.fori_loop(0, nchunk // 2, pair_body, 0)
        if nchunk % 2:
            do_chunk(nchunk - 1, (nchunk - 1) % 2, False, False)

    return facetab_kernel


def _make_shade_kernel(np_pix, f_pad, pix_per_batch):
    nchunk = np_pix // (_NW * _PCB)
    mesh = plsc.VectorSubcoreMesh(core_axis_name="c", subcore_axis_name="s")
    out = jax.ShapeDtypeStruct((np_pix,), jnp.float32)

    @functools.partial(
        pl.kernel,
        out_type=(out, out, out),
        mesh=mesh,
        compiler_params=_PARAMS,
        scratch_types=[
            pltpu.VMEM((_PCB * 3,), jnp.int32),
            pltpu.VMEM((_PCB * 3,), jnp.int32),
            pltpu.VMEM((_PCB * 9,), jnp.float32),
            pltpu.VMEM((_PCB * 9,), jnp.float32),
            pltpu.VMEM((_PCB * 3, 8), jnp.float32),
            pltpu.VMEM((_PCB * 3, 8), jnp.float32),
            pltpu.VMEM((_PCB,), jnp.float32),
            pltpu.VMEM((_PCB,), jnp.float32),
            pltpu.VMEM((_PCB,), jnp.float32),
            pltpu.VMEM((16,), jnp.float32),
            pltpu.SemaphoreType.DMA,
            pltpu.SemaphoreType.DMA,
            pltpu.SemaphoreType.DMA,
            pltpu.SemaphoreType.DMA,
        ],
    )
    def shade_kernel(ftab_hbm, p2f_hbm, bary_hbm, cam_hbm, o0_hbm, o1_hbm,
                     o2_hbm, idx0, idx1, bry0, bry1, rows0, rows1, o0_v, o1_v,
                     o2_v, cam_v, si0, si1, sg0, sg1):
        idxs, brys, rows = (idx0, idx1), (bry0, bry1), (rows0, rows1)
        sin, sg = (si0, si1), (sg0, sg1)
        wid = lax.axis_index("c") * _NS + lax.axis_index("s")
        iota = lax.iota(jnp.int32, _L)
        tec_base = wid * (nchunk * _PCB)
        pltpu.sync_copy(cam_hbm, cam_v)
        outs = (o0_v, o1_v, o2_v)
        out_hbms = (o0_hbm, o1_hbm, o2_hbm)

        def start_in(ch, b):
            pbase = tec_base + ch * _PCB
            n = pbase // pix_per_batch
            pp = pbase - n * pix_per_batch
            for k in range(3):
                pltpu.async_copy(
                    p2f_hbm.at[pl.ds((n * 3 + k) * pix_per_batch + pp, _PCB)],
                    idxs[b].at[pl.ds(k * _PCB, _PCB)], sin[b])
            for kc in range(9):
                pltpu.async_copy(
                    bary_hbm.at[pl.ds((n * 9 + kc) * pix_per_batch + pp,
                                      _PCB)],
                    brys[b].at[pl.ds(kc * _PCB, _PCB)], sin[b])

        def wait_in(b):
            pltpu.make_async_copy(p2f_hbm.at[pl.ds(0, _PCB * 3)],
                                  idxs[b], sin[b]).wait()
            pltpu.make_async_copy(bary_hbm.at[pl.ds(0, _PCB * 9)],
                                  brys[b], sin[b]).wait()

        half = (_PCB * 3) // 2

        def start_gather(b):
            pltpu.async_copy(ftab_hbm.at[idxs[b].at[pl.ds(0, half)]],
                             rows[b].at[pl.ds(0, half)], sg[b])
            pltpu.async_copy(ftab_hbm.at[idxs[b].at[pl.ds(half, half)]],
                             rows[b].at[pl.ds(half, half)], sg[b])

        def wait_gather(b):
            pltpu.make_async_copy(ftab_hbm.at[idxs[b]], rows[b], sg[b]).wait()

        def do_chunk(ch, b, pred_next, pred_next2):
            q = 1 - b
            pbase = tec_base + ch * _PCB
            bidx = pbase // pix_per_batch
            zero16 = jnp.zeros((_L,), jnp.int32)
            cam = [plsc.load_gather(cam_v, [zero16 + (bidx * 3 + m)])
                   for m in range(3)]

            def _next():
                wait_in(q)
                start_gather(q)
            _maybe(pred_next, _next)
            wait_gather(b)

            def group_body(g, c2):
                for k in range(3):
                    row = k * _PCB + g * 16 + iota
                    w = [plsc.load_gather(rows[b], [row, _col(wi)])
                         for wi in range(6)]
                    v0x, v0y = _unpack2(w[0])
                    v0z, v1x = _unpack2(w[1])
                    v1y, v1z = _unpack2(w[2])
                    v2x, v2y = _unpack2(w[3])
                    v2z, n0 = _unpack2(w[4])
                    n1, n2 = _unpack2(w[5])
                    vv = [[v0x, v0y, v0z], [v1x, v1y, v1z], [v2x, v2y, v2z]]
                    nn = [n0, n1, n2]
                    bb = [brys[b][pl.ds((k * 3 + j) * _PCB + g * 16, 16)]
                          for j in range(3)]
                    pts = [bb[0] * vv[0][m] + bb[1] * vv[1][m]
                           + bb[2] * vv[2][m] for m in range(3)]
                    view = [pts[m] - cam[m] for m in range(3)]
                    len2 = jnp.maximum(view[0] * view[0] + view[1] * view[1]
                                       + view[2] * view[2], 1e-24)
                    r = _rsqrt(len2, steps=2)
                    d = (nn[0] * view[0] + nn[1] * view[1]
                         + nn[2] * view[2]) * r
                    outs[k][pl.ds(g * 16, 16)] = d
                return c2

            lax.fori_loop(0, _PCB // _L, group_body, 0)
            for k in range(3):
                pltpu.sync_copy(outs[k], out_hbms[k].at[pl.ds(pbase, _PCB)])
            _maybe(pred_next2, lambda: start_in(ch + 2, b))

        start_in(0, 0)
        wait_in(0)
        start_gather(0)
        start_in(1, 1)

        def pair_body(cp, carry):
            for b in (0, 1):
                ch = cp * 2 + b
                do_chunk(ch, b, ch + 1 < nchunk, ch + 2 < nchunk)
            return carry

        lax.fori_loop(0, nchunk // 2, pair_body, 0)
        if nchunk % 2:
            do_chunk(nchunk - 1, (nchunk - 1) % 2, False, False)

    return shade_kernel


def kernel(pix_to_face, bary_coords, verts, faces, cam_origin):
    n, h, w, k = pix_to_face.shape
    np_pix = n * h * w
    v_cnt = verts.shape[0]
    f_cnt = faces.shape[0]
    align = _NW * _FCB
    f_pad = ((f_cnt + align - 1) // align) * align

    verts_pad = jnp.pad(verts, ((0, 0), (0, 5)))
    faces_flat = jnp.pad(faces, ((0, f_pad - f_cnt), (0, 0))).T.reshape(-1)
    p2f_flat = pix_to_face.transpose(0, 3, 1, 2).reshape(-1)
    bary_flat = bary_coords.transpose(0, 3, 4, 1, 2).reshape(-1)
    cam_pad = jnp.zeros((16,), jnp.float32).at[: n * 3].set(
        cam_origin.reshape(-1))

    ftab = _make_facetab_kernel(f_pad)(verts_pad, faces_flat)
    o0, o1, o2 = _make_shade_kernel(np_pix, f_pad, h * w)(
        ftab.reshape(f_pad, 8), p2f_flat, bary_flat, cam_pad)
    return tuple(o.reshape(n, h, w, 1) for o in (o0, o1, o2))
